# baseline (speedup 1.0000x reference)
import functools

import jax
import jax.numpy as jnp
from jax import lax
from jax.experimental import pallas as pl
from jax.experimental.pallas import tpu as pltpu
from jax.experimental.pallas import tpu_sc as plsc

EMBED_DIM = 64
LANES = 16
NUM_CORES = 2
NUM_SUBCORES = 16
NUM_WORKERS = NUM_CORES * NUM_SUBCORES
WIN = 128
RING = 12
GRP = 16


def _rsqrt_newton(x):
    i = lax.bitcast_convert_type(x, jnp.int32)
    i = jnp.int32(0x5F3759DF) - lax.shift_right_logical(i, 1)
    y = lax.bitcast_convert_type(i, jnp.float32)
    for _ in range(3):
        y = y * (jnp.float32(1.5) - jnp.float32(0.5) * x * y * y)
    return y


def kernel(nodes, table):
    batch = nodes.shape[0]
    n_nodes = table.shape[0]
    b_per_w = batch // NUM_WORKERS
    table_t = table.T
    tail_lo = (n_nodes // WIN) * WIN
    tail_n = n_nodes - tail_lo
    tail_t = table_t[:, tail_lo:]

    mesh = plsc.VectorSubcoreMesh(core_axis_name="c", subcore_axis_name="s")

    @functools.partial(
        pl.kernel,
        mesh=mesh,
        out_type=jax.ShapeDtypeStruct((EMBED_DIM, batch), jnp.float32),
        scratch_types=[
            pltpu.VMEM((b_per_w + 2 * GRP,), jnp.int32),
            pltpu.VMEM((RING, EMBED_DIM, WIN), jnp.float32),
            pltpu.VMEM((EMBED_DIM, tail_n), jnp.float32),
            pltpu.VMEM((2, EMBED_DIM, WIN), jnp.float32),
            pltpu.SemaphoreType.DMA,
            pltpu.SemaphoreType.DMA,
        ],
        compiler_params=pltpu.CompilerParams(
            needs_layout_passes=False, use_tc_tiling_on_sc=True
        ),
    )
    def sc_kernel(
        nodes_hbm, table_hbm, tail_hbm, out_hbm, idx_vm, win_v, tail_v,
        chunk_v, sem, ring_sem,
    ):
        wid = lax.axis_index("s") * NUM_CORES + lax.axis_index("c")
        base = wid * b_per_w
        pltpu.sync_copy(nodes_hbm.at[pl.ds(base, b_per_w)],
                        idx_vm.at[pl.ds(0, b_per_w)])
        pltpu.sync_copy(tail_hbm, tail_v)

        lane = lax.iota(jnp.int32, LANES)

        def win_copy(n, slot):
            nc = jnp.minimum(n, jnp.int32(tail_lo - 1))
            w0 = (nc // WIN) * WIN
            return pltpu.async_copy(
                table_hbm.at[:, pl.ds(w0, WIN)], win_v.at[slot], ring_sem
            )

        head0 = idx_vm[pl.ds(0, LANES)]
        for j in range(RING):
            win_copy(head0[j], j)

        n_chunks = b_per_w // WIN
        grp_per_chunk = WIN // GRP

        def out_wait():
            pltpu.make_async_copy(
                chunk_v.at[0], out_hbm.at[:, pl.ds(base, WIN)], sem
            ).wait()

        def pick_body(g, carry):
            vec = idx_vm[pl.ds(g * GRP, LANES)]
            nxt = idx_vm[pl.ds((g + 1) * GRP, LANES)]
            chunk_id = g // grp_per_chunk
            cbuf = lax.rem(chunk_id, jnp.int32(2))
            goff = GRP * lax.rem(g, jnp.int32(grp_per_chunk))

            @pl.when(
                jnp.logical_and(
                    lax.rem(g, jnp.int32(grp_per_chunk)) == 0, chunk_id >= 2
                )
            )
            def _():
                out_wait()

            for j in range(GRP):
                n = vec[j]
                i = g * GRP + j
                slot = lax.rem(i, jnp.int32(RING))
                pltpu.make_async_copy(
                    table_hbm.at[:, pl.ds(0, WIN)], win_v.at[slot], ring_sem
                ).wait()
                col = lax.broadcast(lax.rem(n, jnp.int32(WIN)), (LANES,))
                jvec = goff + jnp.int32(j)
                jvec = lax.broadcast(jvec, (LANES,))
                for k in range(EMBED_DIM // LANES):
                    d_idx = lane + k * LANES
                    v = plsc.load_gather(win_v.at[slot], [d_idx, col])
                    plsc.store_scatter(chunk_v.at[cbuf], [d_idx, jvec], v)

                @pl.when(n >= tail_lo)
                def _():
                    tcol = lax.broadcast(n - jnp.int32(tail_lo), (LANES,))
                    for k in range(EMBED_DIM // LANES):
                        d_idx = lane + k * LANES
                        v = plsc.load_gather(tail_v, [d_idx, tcol])
                        plsc.store_scatter(chunk_v.at[cbuf], [d_idx, jvec], v)

                n_pf = vec[j + RING] if j + RING < GRP else nxt[j + RING - GRP]

                @pl.when(i + RING < b_per_w)
                def _():
                    win_copy(n_pf, slot)

            sl = pl.ds(goff, LANES)
            acc = jnp.zeros((LANES,), jnp.float32)
            for d in range(EMBED_DIM):
                v = chunk_v[cbuf, d, sl]
                acc = acc + v * v
            inv = _rsqrt_newton(jnp.maximum(acc, jnp.float32(1e-24)))
            for d in range(EMBED_DIM):
                chunk_v[cbuf, d, sl] = chunk_v[cbuf, d, sl] * inv

            @pl.when(lax.rem(g, jnp.int32(grp_per_chunk)) == grp_per_chunk - 1)
            def _():
                pltpu.async_copy(
                    chunk_v.at[cbuf],
                    out_hbm.at[:, pl.ds(base + chunk_id * WIN, WIN)],
                    sem,
                )

            return carry

        lax.fori_loop(0, b_per_w // GRP, pick_body, 0)
        for _ in range(2):
            out_wait()

    out_t = sc_kernel(nodes, table_t, tail_t)
    return out_t.T


# speedup vs baseline: 1.0159x; 1.0159x over previous
import functools

import jax
import jax.numpy as jnp
from jax import lax
from jax.experimental import pallas as pl
from jax.experimental.pallas import tpu as pltpu
from jax.experimental.pallas import tpu_sc as plsc

EMBED_DIM = 64
LANES = 16
NUM_CORES = 2
NUM_SUBCORES = 16
NUM_WORKERS = NUM_CORES * NUM_SUBCORES
WIN = 128
RING = 8
GRP = 16


def _rsqrt_newton(x):
    i = lax.bitcast_convert_type(x, jnp.int32)
    i = jnp.int32(0x5F3759DF) - lax.shift_right_logical(i, 1)
    y = lax.bitcast_convert_type(i, jnp.float32)
    for _ in range(3):
        y = y * (jnp.float32(1.5) - jnp.float32(0.5) * x * y * y)
    return y


def kernel(nodes, table):
    batch = nodes.shape[0]
    n_nodes = table.shape[0]
    b_per_w = batch // NUM_WORKERS
    table_t = table.T
    tail_lo = (n_nodes // WIN) * WIN
    tail_n = n_nodes - tail_lo
    tail_t = table_t[:, tail_lo:]

    mesh = plsc.VectorSubcoreMesh(core_axis_name="c", subcore_axis_name="s")

    @functools.partial(
        pl.kernel,
        mesh=mesh,
        out_type=jax.ShapeDtypeStruct((EMBED_DIM, batch), jnp.float32),
        scratch_types=[
            pltpu.VMEM((b_per_w + 2 * GRP,), jnp.int32),
            pltpu.VMEM((RING, EMBED_DIM, WIN), jnp.float32),
            pltpu.VMEM((EMBED_DIM, tail_n), jnp.float32),
            pltpu.VMEM((EMBED_DIM, b_per_w), jnp.float32),
            pltpu.SemaphoreType.DMA,
        ],
        compiler_params=pltpu.CompilerParams(
            needs_layout_passes=False, use_tc_tiling_on_sc=True
        ),
    )
    def sc_kernel(
        nodes_hbm, table_hbm, tail_hbm, out_hbm, idx_vm, win_v, tail_v,
        cols_v, ring_sem,
    ):
        wid = lax.axis_index("s") * NUM_CORES + lax.axis_index("c")
        base = wid * b_per_w
        pltpu.sync_copy(nodes_hbm.at[pl.ds(base, b_per_w)],
                        idx_vm.at[pl.ds(0, b_per_w)])
        pltpu.sync_copy(tail_hbm, tail_v)

        lane = lax.iota(jnp.int32, LANES)

        def win_copy(n, slot):
            nc = jnp.minimum(n, jnp.int32(tail_lo - 1))
            w0 = (nc // WIN) * WIN
            return pltpu.async_copy(
                table_hbm.at[:, pl.ds(w0, WIN)], win_v.at[slot], ring_sem
            )

        head0 = idx_vm[pl.ds(0, LANES)]
        for j in range(RING):
            win_copy(head0[j], j)

        def pick_body(g, carry):
            vec = idx_vm[pl.ds(g * GRP, LANES)]
            nxt = idx_vm[pl.ds((g + 1) * GRP, LANES)]
            for j in range(GRP):
                n = vec[j]
                i = g * GRP + j
                slot = lax.rem(i, jnp.int32(RING))
                pltpu.make_async_copy(
                    table_hbm.at[:, pl.ds(0, WIN)], win_v.at[slot], ring_sem
                ).wait()
                col = lax.broadcast(lax.rem(n, jnp.int32(WIN)), (LANES,))
                ivec = lax.broadcast(i, (LANES,))
                for k in range(EMBED_DIM // LANES):
                    d_idx = lane + k * LANES
                    v = plsc.load_gather(win_v.at[slot], [d_idx, col])
                    plsc.store_scatter(cols_v, [d_idx, ivec], v)

                @pl.when(n >= tail_lo)
                def _():
                    tcol = lax.broadcast(n - jnp.int32(tail_lo), (LANES,))
                    for k in range(EMBED_DIM // LANES):
                        d_idx = lane + k * LANES
                        v = plsc.load_gather(tail_v, [d_idx, tcol])
                        plsc.store_scatter(cols_v, [d_idx, ivec], v)

                n_pf = vec[j + RING] if j + RING < GRP else nxt[j + RING - GRP]

                @pl.when(i + RING < b_per_w)
                def _():
                    win_copy(n_pf, slot)

            return carry

        lax.fori_loop(0, b_per_w // GRP, pick_body, 0)

        def group_body(g, carry):
            sl = pl.ds(g * LANES, LANES)
            acc = jnp.zeros((LANES,), jnp.float32)
            for d in range(EMBED_DIM):
                v = cols_v[d, sl]
                acc = acc + v * v
            inv = _rsqrt_newton(jnp.maximum(acc, jnp.float32(1e-24)))
            for d in range(EMBED_DIM):
                cols_v[d, sl] = cols_v[d, sl] * inv
            return carry

        lax.fori_loop(0, b_per_w // LANES, group_body, 0)
        pltpu.sync_copy(cols_v, out_hbm.at[:, pl.ds(base, b_per_w)])

    out_t = sc_kernel(nodes, table_t, tail_t)
    return out_t.T
